# baseline (device time: 12596 ns/iter reference)
import jax
import jax.numpy as jnp
from jax import lax
from jax.experimental import pallas as pl
from jax.experimental.pallas import tpu as pltpu

N_Z = 4
BM = 128


def kernel(x, dy, gamma):
    m, d = x.shape
    n_blocks = m // BM

    def body(x_ref, dy_ref, out_ref, acc_ref, comm_ref, send_sems, recv_sems):
        i = pl.program_id(0)
        my_x = lax.axis_index("x")
        my_y = lax.axis_index("y")
        my_z = lax.axis_index("z")

        @pl.when(i == 0)
        def _():
            barrier_sem = pltpu.get_barrier_semaphore()
            for dz in range(1, N_Z):
                tz = (my_z + dz) % N_Z
                pl.semaphore_signal(
                    barrier_sem,
                    inc=1,
                    device_id=(my_x, my_y, tz),
                    device_id_type=pl.DeviceIdType.MESH,
                )

        xv = x_ref[:, :]
        dyv = dy_ref[:, :]
        mu = jnp.mean(xv, axis=1, keepdims=True)
        xc = xv - mu
        var = jnp.mean(xc * xc, axis=1, keepdims=True)
        rstd = lax.rsqrt(var + 1e-5)
        dgamma = jnp.sum(dyv * (xc * rstd), axis=0, keepdims=True)
        dbeta = jnp.sum(dyv, axis=0, keepdims=True)
        part = jnp.concatenate([dgamma, dbeta], axis=0)

        @pl.when(i == 0)
        def _():
            acc_ref[:, :] = part

        @pl.when(i > 0)
        def _():
            acc_ref[:, :] = acc_ref[:, :] + part

        @pl.when(i == n_blocks - 1)
        def _():
            barrier_sem = pltpu.get_barrier_semaphore()
            pl.semaphore_wait(barrier_sem, N_Z - 1)
            rdmas = []
            for dz in range(1, N_Z):
                tz = (my_z + dz) % N_Z
                rdma = pltpu.make_async_remote_copy(
                    src_ref=acc_ref,
                    dst_ref=comm_ref.at[dz - 1],
                    send_sem=send_sems.at[dz - 1],
                    recv_sem=recv_sems.at[dz - 1],
                    device_id=(my_x, my_y, tz),
                    device_id_type=pl.DeviceIdType.MESH,
                )
                rdma.start()
                rdmas.append(rdma)
            for rdma in rdmas:
                rdma.wait_send()
            for rdma in rdmas:
                rdma.wait_recv()
            out_ref[:, :] = (
                acc_ref[:, :] + comm_ref[0] + comm_ref[1] + comm_ref[2]
            )

    return pl.pallas_call(
        body,
        grid=(n_blocks,),
        out_shape=jax.ShapeDtypeStruct((2, d), jnp.float32),
        in_specs=[
            pl.BlockSpec((BM, d), lambda i: (i, 0)),
            pl.BlockSpec((BM, d), lambda i: (i, 0)),
        ],
        out_specs=pl.BlockSpec((2, d), lambda i: (0, 0)),
        scratch_shapes=[
            pltpu.VMEM((2, d), jnp.float32),
            pltpu.VMEM((N_Z - 1, 2, d), jnp.float32),
            pltpu.SemaphoreType.DMA((N_Z - 1,)),
            pltpu.SemaphoreType.DMA((N_Z - 1,)),
        ],
        compiler_params=pltpu.CompilerParams(
            collective_id=0,
            dimension_semantics=("arbitrary",),
        ),
    )(x, dy)


# device time: 11661 ns/iter; 1.0802x vs baseline; 1.0802x over previous
import jax
import jax.numpy as jnp
from jax import lax
from jax.experimental import pallas as pl
from jax.experimental.pallas import tpu as pltpu

N_Z = 4
BM = 512


def kernel(x, dy, gamma):
    m, d = x.shape
    n_blocks = m // BM

    def body(x_ref, dy_ref, out_ref, acc_ref, comm_ref, send_sems, recv_sems):
        i = pl.program_id(0)
        my_x = lax.axis_index("x")
        my_y = lax.axis_index("y")
        my_z = lax.axis_index("z")

        @pl.when(i == 0)
        def _():
            barrier_sem = pltpu.get_barrier_semaphore()
            for dz in range(1, N_Z):
                tz = (my_z + dz) % N_Z
                pl.semaphore_signal(
                    barrier_sem,
                    inc=1,
                    device_id=(my_x, my_y, tz),
                    device_id_type=pl.DeviceIdType.MESH,
                )

        xv = x_ref[:, :]
        dyv = dy_ref[:, :]
        mu = jnp.mean(xv, axis=1, keepdims=True)
        xc = xv - mu
        var = jnp.mean(xc * xc, axis=1, keepdims=True)
        rstd = lax.rsqrt(var + 1e-5)
        dgamma = jnp.sum(dyv * (xc * rstd), axis=0, keepdims=True)
        dbeta = jnp.sum(dyv, axis=0, keepdims=True)
        part = jnp.concatenate([dgamma, dbeta], axis=0)

        @pl.when(i == 0)
        def _():
            acc_ref[:, :] = part

        @pl.when(i > 0)
        def _():
            acc_ref[:, :] = acc_ref[:, :] + part

        @pl.when(i == n_blocks - 1)
        def _():
            barrier_sem = pltpu.get_barrier_semaphore()
            pl.semaphore_wait(barrier_sem, N_Z - 1)
            rdmas = []
            for dz in range(1, N_Z):
                tz = (my_z + dz) % N_Z
                rdma = pltpu.make_async_remote_copy(
                    src_ref=acc_ref,
                    dst_ref=comm_ref.at[dz - 1],
                    send_sem=send_sems.at[dz - 1],
                    recv_sem=recv_sems.at[dz - 1],
                    device_id=(my_x, my_y, tz),
                    device_id_type=pl.DeviceIdType.MESH,
                )
                rdma.start()
                rdmas.append(rdma)
            for rdma in rdmas:
                rdma.wait_send()
            for rdma in rdmas:
                rdma.wait_recv()
            out_ref[:, :] = (
                acc_ref[:, :] + comm_ref[0] + comm_ref[1] + comm_ref[2]
            )

    return pl.pallas_call(
        body,
        grid=(n_blocks,),
        out_shape=jax.ShapeDtypeStruct((2, d), jnp.float32),
        in_specs=[
            pl.BlockSpec((BM, d), lambda i: (i, 0)),
            pl.BlockSpec((BM, d), lambda i: (i, 0)),
        ],
        out_specs=pl.BlockSpec((2, d), lambda i: (0, 0)),
        scratch_shapes=[
            pltpu.VMEM((2, d), jnp.float32),
            pltpu.VMEM((N_Z - 1, 2, d), jnp.float32),
            pltpu.SemaphoreType.DMA((N_Z - 1,)),
            pltpu.SemaphoreType.DMA((N_Z - 1,)),
        ],
        compiler_params=pltpu.CompilerParams(
            collective_id=0,
            dimension_semantics=("arbitrary",),
        ),
    )(x, dy)
